# Initial kernel scaffold; baseline (speedup 1.0000x reference)
import jax
import jax.numpy as jnp
from jax.experimental import pallas as pl


def kernel(x, pos_emb):
    raise NotImplementedError("write your pallas kernel here")


# speedup vs baseline: 16.2757x; 16.2757x over previous
import functools

import jax
import jax.numpy as jnp
from jax.experimental import pallas as pl

_KNN = 32


def _float_keys(s):
    u = jax.lax.bitcast_convert_type(s, jnp.uint32)
    flip = jnp.where(
        (u >> 31) == jnp.uint32(1),
        jnp.uint32(0xFFFFFFFF),
        jnp.uint32(0x80000000),
    )
    return u ^ flip


def _graph_kernel(x_ref, pos_ref, out_ref, *, block_rows: int, knn: int):
    i = pl.program_id(1)
    xb = x_ref[0] + pos_ref[...]
    norm = jnp.sqrt(jnp.sum(xb * xb, axis=-1, keepdims=True))
    xn = xb / (norm + 1e-07)
    qx = x_ref[0, pl.ds(i * block_rows, block_rows), :] + pos_ref[
        pl.ds(i * block_rows, block_rows), :
    ]
    qnorm = jnp.sqrt(jnp.sum(qx * qx, axis=-1, keepdims=True))
    q = qx / (qnorm + 1e-07)
    s = jax.lax.dot_general(
        q, xn, (((1,), (1,)), ((), ())), preferred_element_type=jnp.float32
    )
    key = _float_keys(s)
    t = jnp.zeros((block_rows, 1), dtype=jnp.uint32)
    for step in range(32):
        bit = jnp.uint32(1 << (31 - step))
        cand = t | bit
        cnt = jnp.sum((key >= cand).astype(jnp.int32), axis=1, keepdims=True)
        t = jnp.where(cnt >= knn, cand, t)
    out_ref[0] = jnp.where(key >= t, s, 0.0)


def kernel(x, pos_emb):
    b, n, d = x.shape
    block_rows = 256
    grid = (b, n // block_rows)
    return pl.pallas_call(
        functools.partial(_graph_kernel, block_rows=block_rows, knn=_KNN),
        grid=grid,
        in_specs=[
            pl.BlockSpec((1, n, d), lambda bi, i: (bi, 0, 0)),
            pl.BlockSpec((n, d), lambda bi, i: (0, 0)),
        ],
        out_specs=pl.BlockSpec((1, block_rows, n), lambda bi, i: (bi, i, 0)),
        out_shape=jax.ShapeDtypeStruct((b, n, n), jnp.float32),
    )(x, pos_emb)
